# baseline (speedup 1.0000x reference)
import jax
import jax.numpy as jnp
from jax.experimental import pallas as pl
from jax.experimental.pallas import tpu as pltpu

N_WIN = 7
NUM_HEADS = 8
TOPK = 4
DIM = 192
HD = DIM // NUM_HEADS
WS = 16
W2 = WS * WS
P2 = N_WIN * N_WIN
SCALE = DIM ** -0.5
ROWS = 16
IMG = N_WIN * WS


def _qkv_kernel(x_ref, w_ref, b_ref, q_ref, kv_ref, vpad_ref,
                qwin_ref, kwin_ref):
    i = pl.program_id(0)
    j = pl.program_id(1)
    xw = x_ref[...].reshape(W2, DIM)
    q = (jnp.dot(xw, w_ref[:, :DIM], preferred_element_type=jnp.float32)
         + b_ref[:, :DIM]) * SCALE
    kv = (jnp.dot(xw, w_ref[:, DIM:], preferred_element_type=jnp.float32)
          + b_ref[:, DIM:])
    q_ref[0] = q
    kv_ref[0] = kv

    @pl.when((i == 0) & (j == 0))
    def _zero():
        vpad_ref[...] = jnp.zeros(vpad_ref.shape, jnp.float32)

    vpad_ref[pl.ds(i * WS + 2, WS), pl.ds(j * WS + 8, WS), :] = (
        kv[:, DIM:].reshape(WS, WS, DIM))
    qwin_ref[0] = jnp.mean(q, axis=0, keepdims=True)
    kwin_ref[0] = jnp.mean(kv[:, :DIM], axis=0, keepdims=True)


def _router_kernel(qw_ref, kw_ref, o0, o1, o2, o3):
    qw = qw_ref[...].reshape(P2, DIM)
    kw = kw_ref[...].reshape(P2, DIM)
    logits = jax.lax.dot_general(qw, kw, (((1,), (1,)), ((), ())),
                                 preferred_element_type=jnp.float32)
    cols = jax.lax.broadcasted_iota(jnp.int32, (P2, P2), 1)
    outs = (o0, o1, o2, o3)
    for t in range(TOPK):
        m = jnp.max(logits, axis=1, keepdims=True)
        idx = jnp.min(jnp.where(logits == m, cols, P2), axis=1, keepdims=True)
        outs[t][...] = idx
        logits = jnp.where(cols == idx, -jnp.inf, logits)


def _attn_kernel(i0, i1, i2, i3, q_ref, kv_ref, o_ref, acc_ref, m_ref, s_ref):
    step = pl.program_id(0)
    t = step % TOPK

    @pl.when(t == 0)
    def _reset():
        m_ref[...] = jnp.full(m_ref.shape, -jnp.inf, jnp.float32)
        s_ref[...] = jnp.zeros(s_ref.shape, jnp.float32)
        acc_ref[...] = jnp.zeros(acc_ref.shape, jnp.float32)

    q = q_ref[0]
    kv = kv_ref[0]
    for h in range(NUM_HEADS):
        lo, hi = h * HD, (h + 1) * HD
        lg = jax.lax.dot_general(q[:, lo:hi], kv[:, lo:hi],
                                 (((1,), (1,)), ((), ())),
                                 preferred_element_type=jnp.float32)
        m_old = m_ref[:, h:h + 1]
        m_new = jnp.maximum(m_old, jnp.max(lg, axis=1, keepdims=True))
        corr = jnp.exp(m_old - m_new)
        p = jnp.exp(lg - m_new)
        s_ref[:, h:h + 1] = s_ref[:, h:h + 1] * corr + jnp.sum(
            p, axis=1, keepdims=True)
        pv = jax.lax.dot_general(p.astype(jnp.bfloat16),
                                 kv[:, DIM + lo:DIM + hi].astype(jnp.bfloat16),
                                 (((1,), (0,)), ((), ())),
                                 preferred_element_type=jnp.float32)
        acc_ref[:, lo:hi] = acc_ref[:, lo:hi] * corr + pv
        m_ref[:, h:h + 1] = m_new

    @pl.when(t == TOPK - 1)
    def _finish():
        outs = [acc_ref[:, h * HD:(h + 1) * HD] / s_ref[:, h:h + 1]
                for h in range(NUM_HEADS)]
        o_ref[...] = jnp.concatenate(outs, axis=1).reshape(WS, WS, DIM)


def _out_kernel(attn_ref, vpad_ref, lw_ref, lb_ref, wo_ref, wob_ref, o_ref):
    i = pl.program_id(0)
    acc = attn_ref[...]
    for di in range(5):
        for dj in range(5):
            w = lw_ref[di * 5 + dj:di * 5 + dj + 1, :].reshape(1, 1, DIM)
            acc = acc + vpad_ref[pl.ds(i * ROWS + di, ROWS),
                                 pl.ds(dj + 6, IMG), :] * w
    acc = acc + lb_ref[...].reshape(1, 1, DIM)
    y = jnp.dot(acc.reshape(ROWS * IMG, DIM), wo_ref[...],
                preferred_element_type=jnp.float32) + wob_ref[...]
    o_ref[...] = y.reshape(ROWS, IMG, DIM)


def kernel(x, qkv_w, qkv_b, wo_w, wo_b, lepe_w, lepe_b):
    B, H, W, C = x.shape
    f32 = jnp.float32
    q, kv, vpad, qwin, kwin = pl.pallas_call(
        _qkv_kernel,
        grid=(N_WIN, N_WIN),
        in_specs=[
            pl.BlockSpec((1, WS, WS, DIM), lambda i, j: (0, i, j, 0)),
            pl.BlockSpec((DIM, 3 * DIM), lambda i, j: (0, 0)),
            pl.BlockSpec((1, 3 * DIM), lambda i, j: (0, 0)),
        ],
        out_specs=[
            pl.BlockSpec((1, W2, DIM), lambda i, j: (i * N_WIN + j, 0, 0)),
            pl.BlockSpec((1, W2, 2 * DIM), lambda i, j: (i * N_WIN + j, 0, 0)),
            pl.BlockSpec((IMG + 4, 128, DIM), lambda i, j: (0, 0, 0)),
            pl.BlockSpec((1, 1, DIM), lambda i, j: (i * N_WIN + j, 0, 0)),
            pl.BlockSpec((1, 1, DIM), lambda i, j: (i * N_WIN + j, 0, 0)),
        ],
        out_shape=[
            jax.ShapeDtypeStruct((P2, W2, DIM), f32),
            jax.ShapeDtypeStruct((P2, W2, 2 * DIM), f32),
            jax.ShapeDtypeStruct((IMG + 4, 128, DIM), f32),
            jax.ShapeDtypeStruct((P2, 1, DIM), f32),
            jax.ShapeDtypeStruct((P2, 1, DIM), f32),
        ],
    )(x, qkv_w, qkv_b.reshape(1, 3 * DIM))

    o0, o1, o2, o3 = pl.pallas_call(
        _router_kernel,
        out_shape=[jax.ShapeDtypeStruct((P2, 1), jnp.int32)] * TOPK,
    )(qwin, kwin)

    def _kv_imap(s, i0, i1, i2, i3):
        p = s // TOPK
        t = s % TOPK
        idx = jnp.where(t == 0, i0[p, 0],
              jnp.where(t == 1, i1[p, 0],
              jnp.where(t == 2, i2[p, 0], i3[p, 0])))
        return (idx, 0, 0)

    attn_img = pl.pallas_call(
        _attn_kernel,
        grid_spec=pltpu.PrefetchScalarGridSpec(
            num_scalar_prefetch=4,
            grid=(P2 * TOPK,),
            in_specs=[
                pl.BlockSpec((1, W2, DIM),
                             lambda s, i0, i1, i2, i3: (s // TOPK, 0, 0)),
                pl.BlockSpec((1, W2, 2 * DIM), _kv_imap),
            ],
            out_specs=pl.BlockSpec(
                (WS, WS, DIM),
                lambda s, i0, i1, i2, i3: (s // (TOPK * N_WIN),
                                           (s // TOPK) % N_WIN, 0)),
            scratch_shapes=[
                pltpu.VMEM((W2, DIM), f32),
                pltpu.VMEM((W2, NUM_HEADS), f32),
                pltpu.VMEM((W2, NUM_HEADS), f32),
            ],
        ),
        out_shape=jax.ShapeDtypeStruct((IMG, IMG, DIM), f32),
    )(o0, o1, o2, o3, q, kv)

    out = pl.pallas_call(
        _out_kernel,
        grid=(IMG // ROWS,),
        in_specs=[
            pl.BlockSpec((ROWS, IMG, DIM), lambda i: (i, 0, 0)),
            pl.BlockSpec((IMG + 4, 128, DIM), lambda i: (0, 0, 0)),
            pl.BlockSpec((25, DIM), lambda i: (0, 0)),
            pl.BlockSpec((1, DIM), lambda i: (0, 0)),
            pl.BlockSpec((DIM, DIM), lambda i: (0, 0)),
            pl.BlockSpec((1, DIM), lambda i: (0, 0)),
        ],
        out_specs=pl.BlockSpec((ROWS, IMG, DIM), lambda i: (i, 0, 0)),
        out_shape=jax.ShapeDtypeStruct((IMG, IMG, DIM), f32),
    )(attn_img, vpad, lepe_w.reshape(25, DIM), lepe_b.reshape(1, DIM),
      wo_w, wo_b.reshape(1, DIM))

    return out[None]


# speedup vs baseline: 3.3377x; 3.3377x over previous
import jax
import jax.numpy as jnp
from jax.experimental import pallas as pl
from jax.experimental.pallas import tpu as pltpu

N_WIN = 7
NUM_HEADS = 8
TOPK = 4
DIM = 192
HD = DIM // NUM_HEADS
WS = 16
W2 = WS * WS
P2 = N_WIN * N_WIN
SCALE = DIM ** -0.5
ROWS = 16
IMG = N_WIN * WS


def _qkv_kernel(x_ref, w_ref, b_ref, q_ref, kt_ref, vh_ref, vpad_ref,
                qwin_ref, kwin_ref):
    i = pl.program_id(0)
    j = pl.program_id(1)
    xw = x_ref[...].reshape(W2, DIM)
    q = (jnp.dot(xw, w_ref[:, :DIM], preferred_element_type=jnp.float32)
         + b_ref[:, :DIM]) * SCALE
    kv = (jnp.dot(xw, w_ref[:, DIM:], preferred_element_type=jnp.float32)
          + b_ref[:, DIM:])
    k = kv[:, :DIM]
    v = kv[:, DIM:]
    q_ref[0] = q
    kt_ref[0] = k.T
    vh_ref[0] = v.reshape(W2, NUM_HEADS, HD).transpose(1, 0, 2)

    @pl.when((i == 0) & (j == 0))
    def _zero():
        vpad_ref[...] = jnp.zeros(vpad_ref.shape, jnp.float32)

    vpad_ref[pl.ds(i * WS + 2, WS), pl.ds(j * WS + 8, WS), :] = (
        v.reshape(WS, WS, DIM))
    qwin_ref[0] = jnp.mean(q, axis=0, keepdims=True)
    kwin_ref[0] = jnp.mean(k, axis=0, keepdims=True)


def _router_kernel(qw_ref, kw_ref, o0, o1, o2, o3):
    qw = qw_ref[...].reshape(P2, DIM)
    kw = kw_ref[...].reshape(P2, DIM)
    logits = jax.lax.dot_general(qw, kw, (((1,), (1,)), ((), ())),
                                 preferred_element_type=jnp.float32)
    cols = jax.lax.broadcasted_iota(jnp.int32, (P2, P2), 1)
    outs = (o0, o1, o2, o3)
    for t in range(TOPK):
        m = jnp.max(logits, axis=1, keepdims=True)
        idx = jnp.min(jnp.where(logits == m, cols, P2), axis=1, keepdims=True)
        outs[t][...] = idx
        logits = jnp.where(cols == idx, -jnp.inf, logits)


def _attn_kernel(i0, i1, i2, i3, q_ref, kt0, kt1, kt2, kt3,
                 vh0, vh1, vh2, vh3, o_ref):
    kts = (kt0, kt1, kt2, kt3)
    vhs = (vh0, vh1, vh2, vh3)
    q = q_ref[0]
    outs = []
    for h in range(NUM_HEADS):
        lo, hi = h * HD, (h + 1) * HD
        qh = q[:, lo:hi]
        lgs = [
            jax.lax.dot_general(qh, kts[t][0][lo:hi, :],
                                (((1,), (0,)), ((), ())),
                                preferred_element_type=jnp.float32)
            for t in range(TOPK)]
        m = lgs[0].max(axis=1, keepdims=True)
        for t in range(1, TOPK):
            m = jnp.maximum(m, lgs[t].max(axis=1, keepdims=True))
        s = None
        oh = None
        for t in range(TOPK):
            p = jnp.exp(lgs[t] - m)
            ps = jnp.sum(p, axis=1, keepdims=True)
            s = ps if s is None else s + ps
            c = jax.lax.dot_general(p.astype(jnp.bfloat16),
                                    vhs[t][0, h].astype(jnp.bfloat16),
                                    (((1,), (0,)), ((), ())),
                                    preferred_element_type=jnp.float32)
            oh = c if oh is None else oh + c
        outs.append(oh / s)
    o_ref[...] = jnp.concatenate(outs, axis=1).reshape(WS, WS, DIM)


def _out_kernel(attn_ref, vpad_ref, lw_ref, lb_ref, wo_ref, wob_ref, o_ref):
    i = pl.program_id(0)
    acc = attn_ref[...]
    for di in range(5):
        for dj in range(5):
            w = lw_ref[di * 5 + dj:di * 5 + dj + 1, :].reshape(1, 1, DIM)
            acc = acc + vpad_ref[pl.ds(i * ROWS + di, ROWS),
                                 pl.ds(dj + 6, IMG), :] * w
    acc = acc + lb_ref[...].reshape(1, 1, DIM)
    y = jnp.dot(acc.reshape(ROWS * IMG, DIM), wo_ref[...],
                preferred_element_type=jnp.float32) + wob_ref[...]
    o_ref[...] = y.reshape(ROWS, IMG, DIM)


def kernel(x, qkv_w, qkv_b, wo_w, wo_b, lepe_w, lepe_b):
    B, H, W, C = x.shape
    f32 = jnp.float32
    q, kt, vh, vpad, qwin, kwin = pl.pallas_call(
        _qkv_kernel,
        grid=(N_WIN, N_WIN),
        in_specs=[
            pl.BlockSpec((1, WS, WS, DIM), lambda i, j: (0, i, j, 0)),
            pl.BlockSpec((DIM, 3 * DIM), lambda i, j: (0, 0)),
            pl.BlockSpec((1, 3 * DIM), lambda i, j: (0, 0)),
        ],
        out_specs=[
            pl.BlockSpec((1, W2, DIM), lambda i, j: (i * N_WIN + j, 0, 0)),
            pl.BlockSpec((1, DIM, W2), lambda i, j: (i * N_WIN + j, 0, 0)),
            pl.BlockSpec((1, NUM_HEADS, W2, HD),
                         lambda i, j: (i * N_WIN + j, 0, 0, 0)),
            pl.BlockSpec((IMG + 4, 128, DIM), lambda i, j: (0, 0, 0)),
            pl.BlockSpec((1, 1, DIM), lambda i, j: (i * N_WIN + j, 0, 0)),
            pl.BlockSpec((1, 1, DIM), lambda i, j: (i * N_WIN + j, 0, 0)),
        ],
        out_shape=[
            jax.ShapeDtypeStruct((P2, W2, DIM), f32),
            jax.ShapeDtypeStruct((P2, DIM, W2), f32),
            jax.ShapeDtypeStruct((P2, NUM_HEADS, W2, HD), f32),
            jax.ShapeDtypeStruct((IMG + 4, 128, DIM), f32),
            jax.ShapeDtypeStruct((P2, 1, DIM), f32),
            jax.ShapeDtypeStruct((P2, 1, DIM), f32),
        ],
    )(x, qkv_w, qkv_b.reshape(1, 3 * DIM))

    o0, o1, o2, o3 = pl.pallas_call(
        _router_kernel,
        out_shape=[jax.ShapeDtypeStruct((P2, 1), jnp.int32)] * TOPK,
    )(qwin, kwin)

    def _kt_spec(t):
        return pl.BlockSpec(
            (1, DIM, W2),
            lambda p, i0, i1, i2, i3, t=t: ((i0, i1, i2, i3)[t][p, 0], 0, 0))

    def _vh_spec(t):
        return pl.BlockSpec(
            (1, NUM_HEADS, W2, HD),
            lambda p, i0, i1, i2, i3, t=t: ((i0, i1, i2, i3)[t][p, 0], 0, 0, 0))

    attn_img = pl.pallas_call(
        _attn_kernel,
        grid_spec=pltpu.PrefetchScalarGridSpec(
            num_scalar_prefetch=4,
            grid=(P2,),
            in_specs=[
                pl.BlockSpec((1, W2, DIM),
                             lambda p, i0, i1, i2, i3: (p, 0, 0)),
                _kt_spec(0), _kt_spec(1), _kt_spec(2), _kt_spec(3),
                _vh_spec(0), _vh_spec(1), _vh_spec(2), _vh_spec(3),
            ],
            out_specs=pl.BlockSpec(
                (WS, WS, DIM),
                lambda p, i0, i1, i2, i3: (p // N_WIN, p % N_WIN, 0)),
        ),
        out_shape=jax.ShapeDtypeStruct((IMG, IMG, DIM), f32),
    )(o0, o1, o2, o3, q, kt, kt, kt, kt, vh, vh, vh, vh)

    out = pl.pallas_call(
        _out_kernel,
        grid=(IMG // ROWS,),
        in_specs=[
            pl.BlockSpec((ROWS, IMG, DIM), lambda i: (i, 0, 0)),
            pl.BlockSpec((IMG + 4, 128, DIM), lambda i: (0, 0, 0)),
            pl.BlockSpec((25, DIM), lambda i: (0, 0)),
            pl.BlockSpec((1, DIM), lambda i: (0, 0)),
            pl.BlockSpec((DIM, DIM), lambda i: (0, 0)),
            pl.BlockSpec((1, DIM), lambda i: (0, 0)),
        ],
        out_specs=pl.BlockSpec((ROWS, IMG, DIM), lambda i: (i, 0, 0)),
        out_shape=jax.ShapeDtypeStruct((IMG, IMG, DIM), f32),
    )(attn_img, vpad, lepe_w.reshape(25, DIM), lepe_b.reshape(1, DIM),
      wo_w, wo_b.reshape(1, DIM))

    return out[None]


# speedup vs baseline: 3.3618x; 1.0072x over previous
import jax
import jax.numpy as jnp
from jax.experimental import pallas as pl
from jax.experimental.pallas import tpu as pltpu

N_WIN = 7
NUM_HEADS = 8
TOPK = 4
DIM = 192
HD = DIM // NUM_HEADS
WS = 16
W2 = WS * WS
P2 = N_WIN * N_WIN
SCALE = DIM ** -0.5
ROWS = 16
IMG = N_WIN * WS


def _qkv_kernel(x_ref, w_ref, b_ref, q_ref, kt_ref, vh_ref, vpad_ref,
                qwin_ref, kwin_ref):
    i = pl.program_id(0)
    j = pl.program_id(1)
    xw = x_ref[...].reshape(W2, DIM)
    q = (jnp.dot(xw, w_ref[:, :DIM], preferred_element_type=jnp.float32)
         + b_ref[:, :DIM]) * SCALE
    kv = (jnp.dot(xw, w_ref[:, DIM:], preferred_element_type=jnp.float32)
          + b_ref[:, DIM:])
    k = kv[:, :DIM]
    v = kv[:, DIM:]
    q_ref[0] = q
    kt_ref[0] = k.T
    vh = jnp.concatenate(
        [v.reshape(W2, NUM_HEADS, HD).transpose(1, 0, 2),
         jnp.ones((NUM_HEADS, W2, 1), jnp.float32)], axis=2)
    vh_ref[0] = vh

    @pl.when((i == 0) & (j == 0))
    def _zero():
        vpad_ref[...] = jnp.zeros(vpad_ref.shape, jnp.float32)

    vpad_ref[pl.ds(i * WS + 2, WS), pl.ds(j * WS + 8, WS), :] = (
        v.reshape(WS, WS, DIM))
    qwin_ref[0] = jnp.mean(q, axis=0, keepdims=True)
    kwin_ref[0] = jnp.mean(k, axis=0, keepdims=True)


def _router_kernel(qw_ref, kw_ref, o0, o1, o2, o3):
    qw = qw_ref[...].reshape(P2, DIM)
    kw = kw_ref[...].reshape(P2, DIM)
    logits = jax.lax.dot_general(qw, kw, (((1,), (1,)), ((), ())),
                                 preferred_element_type=jnp.float32)
    cols = jax.lax.broadcasted_iota(jnp.int32, (P2, P2), 1)
    outs = (o0, o1, o2, o3)
    for t in range(TOPK):
        m = jnp.max(logits, axis=1, keepdims=True)
        idx = jnp.min(jnp.where(logits == m, cols, P2), axis=1, keepdims=True)
        outs[t][...] = idx
        logits = jnp.where(cols == idx, -jnp.inf, logits)


def _attn_kernel(i0, i1, i2, i3, q_ref, kt0, kt1, kt2, kt3,
                 vh0, vh1, vh2, vh3, o_ref):
    kts = (kt0, kt1, kt2, kt3)
    vhs = (vh0, vh1, vh2, vh3)
    q = q_ref[0]
    outs = []
    for h in range(NUM_HEADS):
        lo, hi = h * HD, (h + 1) * HD
        qh = q[:, lo:hi]
        lgs = [
            jax.lax.dot_general(qh, kts[t][0][lo:hi, :],
                                (((1,), (0,)), ((), ())),
                                preferred_element_type=jnp.float32)
            for t in range(TOPK)]
        m = lgs[0].max(axis=1, keepdims=True)
        for t in range(1, TOPK):
            m = jnp.maximum(m, lgs[t].max(axis=1, keepdims=True))
        oh = None
        for t in range(TOPK):
            p = jnp.exp(lgs[t] - m)
            c = jax.lax.dot_general(p.astype(jnp.bfloat16),
                                    vhs[t][0, h].astype(jnp.bfloat16),
                                    (((1,), (0,)), ((), ())),
                                    preferred_element_type=jnp.float32)
            oh = c if oh is None else oh + c
        outs.append(oh[:, :HD] / oh[:, HD:HD + 1])
    o_ref[...] = jnp.concatenate(outs, axis=1).reshape(WS, WS, DIM)


def _out_kernel(attn_ref, vpad_ref, lw_ref, lb_ref, wo_ref, wob_ref, o_ref):
    i = pl.program_id(0)
    acc = attn_ref[...]
    for di in range(5):
        for dj in range(5):
            w = lw_ref[di * 5 + dj:di * 5 + dj + 1, :].reshape(1, 1, DIM)
            acc = acc + vpad_ref[pl.ds(i * ROWS + di, ROWS),
                                 pl.ds(dj + 6, IMG), :] * w
    acc = acc + lb_ref[...].reshape(1, 1, DIM)
    y = jnp.dot(acc.reshape(ROWS * IMG, DIM), wo_ref[...],
                preferred_element_type=jnp.float32) + wob_ref[...]
    o_ref[...] = y.reshape(ROWS, IMG, DIM)


def kernel(x, qkv_w, qkv_b, wo_w, wo_b, lepe_w, lepe_b):
    B, H, W, C = x.shape
    f32 = jnp.float32
    q, kt, vh, vpad, qwin, kwin = pl.pallas_call(
        _qkv_kernel,
        grid=(N_WIN, N_WIN),
        in_specs=[
            pl.BlockSpec((1, WS, WS, DIM), lambda i, j: (0, i, j, 0)),
            pl.BlockSpec((DIM, 3 * DIM), lambda i, j: (0, 0)),
            pl.BlockSpec((1, 3 * DIM), lambda i, j: (0, 0)),
        ],
        out_specs=[
            pl.BlockSpec((1, W2, DIM), lambda i, j: (i * N_WIN + j, 0, 0)),
            pl.BlockSpec((1, DIM, W2), lambda i, j: (i * N_WIN + j, 0, 0)),
            pl.BlockSpec((1, NUM_HEADS, W2, HD + 1),
                         lambda i, j: (i * N_WIN + j, 0, 0, 0)),
            pl.BlockSpec((IMG + 4, 128, DIM), lambda i, j: (0, 0, 0)),
            pl.BlockSpec((1, 1, DIM), lambda i, j: (i * N_WIN + j, 0, 0)),
            pl.BlockSpec((1, 1, DIM), lambda i, j: (i * N_WIN + j, 0, 0)),
        ],
        out_shape=[
            jax.ShapeDtypeStruct((P2, W2, DIM), f32),
            jax.ShapeDtypeStruct((P2, DIM, W2), f32),
            jax.ShapeDtypeStruct((P2, NUM_HEADS, W2, HD + 1), f32),
            jax.ShapeDtypeStruct((IMG + 4, 128, DIM), f32),
            jax.ShapeDtypeStruct((P2, 1, DIM), f32),
            jax.ShapeDtypeStruct((P2, 1, DIM), f32),
        ],
    )(x, qkv_w, qkv_b.reshape(1, 3 * DIM))

    o0, o1, o2, o3 = pl.pallas_call(
        _router_kernel,
        out_shape=[jax.ShapeDtypeStruct((P2, 1), jnp.int32)] * TOPK,
    )(qwin, kwin)

    def _kt_spec(t):
        return pl.BlockSpec(
            (1, DIM, W2),
            lambda p, i0, i1, i2, i3, t=t: ((i0, i1, i2, i3)[t][p, 0], 0, 0))

    def _vh_spec(t):
        return pl.BlockSpec(
            (1, NUM_HEADS, W2, HD + 1),
            lambda p, i0, i1, i2, i3, t=t: ((i0, i1, i2, i3)[t][p, 0], 0, 0, 0))

    attn_img = pl.pallas_call(
        _attn_kernel,
        grid_spec=pltpu.PrefetchScalarGridSpec(
            num_scalar_prefetch=4,
            grid=(P2,),
            in_specs=[
                pl.BlockSpec((1, W2, DIM),
                             lambda p, i0, i1, i2, i3: (p, 0, 0)),
                _kt_spec(0), _kt_spec(1), _kt_spec(2), _kt_spec(3),
                _vh_spec(0), _vh_spec(1), _vh_spec(2), _vh_spec(3),
            ],
            out_specs=pl.BlockSpec(
                (WS, WS, DIM),
                lambda p, i0, i1, i2, i3: (p // N_WIN, p % N_WIN, 0)),
        ),
        out_shape=jax.ShapeDtypeStruct((IMG, IMG, DIM), f32),
    )(o0, o1, o2, o3, q, kt, kt, kt, kt, vh, vh, vh, vh)

    out = pl.pallas_call(
        _out_kernel,
        grid=(IMG // ROWS,),
        in_specs=[
            pl.BlockSpec((ROWS, IMG, DIM), lambda i: (i, 0, 0)),
            pl.BlockSpec((IMG + 4, 128, DIM), lambda i: (0, 0, 0)),
            pl.BlockSpec((25, DIM), lambda i: (0, 0)),
            pl.BlockSpec((1, DIM), lambda i: (0, 0)),
            pl.BlockSpec((DIM, DIM), lambda i: (0, 0)),
            pl.BlockSpec((1, DIM), lambda i: (0, 0)),
        ],
        out_specs=pl.BlockSpec((ROWS, IMG, DIM), lambda i: (i, 0, 0)),
        out_shape=jax.ShapeDtypeStruct((IMG, IMG, DIM), f32),
    )(attn_img, vpad, lepe_w.reshape(25, DIM), lepe_b.reshape(1, DIM),
      wo_w, wo_b.reshape(1, DIM))

    return out[None]


# speedup vs baseline: 3.4187x; 1.0169x over previous
import jax
import jax.numpy as jnp
from jax.experimental import pallas as pl
from jax.experimental.pallas import tpu as pltpu

N_WIN = 7
NUM_HEADS = 8
TOPK = 4
DIM = 192
HD = DIM // NUM_HEADS
WS = 16
W2 = WS * WS
P2 = N_WIN * N_WIN
SCALE = DIM ** -0.5
ROWS = 16
IMG = N_WIN * WS


def _qkv_kernel(x_ref, w_ref, b_ref, q_ref, kt_ref, vh_ref, vpad_ref,
                qwin_ref, kwin_ref):
    i = pl.program_id(0)
    j = pl.program_id(1)
    xw = x_ref[...].reshape(W2, DIM)
    q = (jnp.dot(xw, w_ref[:, :DIM], preferred_element_type=jnp.float32)
         + b_ref[:, :DIM]) * SCALE
    kv = (jnp.dot(xw, w_ref[:, DIM:], preferred_element_type=jnp.float32)
          + b_ref[:, DIM:])
    k = kv[:, :DIM]
    v = kv[:, DIM:]
    q_ref[0] = q
    kt_ref[0] = k.T
    vh = jnp.concatenate(
        [v.reshape(W2, NUM_HEADS, HD).transpose(1, 0, 2),
         jnp.ones((NUM_HEADS, W2, 1), jnp.float32)], axis=2)
    vh_ref[0] = vh.astype(jnp.bfloat16)

    @pl.when((i == 0) & (j == 0))
    def _zero():
        vpad_ref[...] = jnp.zeros(vpad_ref.shape, jnp.float32)

    vpad_ref[pl.ds(i * WS + 2, WS), pl.ds(j * WS + 8, WS), :] = (
        v.reshape(WS, WS, DIM))
    qwin_ref[0] = jnp.mean(q, axis=0, keepdims=True)
    kwin_ref[0] = jnp.mean(k, axis=0, keepdims=True)


def _router_kernel(qw_ref, kw_ref, o0, o1, o2, o3):
    qw = qw_ref[...].reshape(P2, DIM)
    kw = kw_ref[...].reshape(P2, DIM)
    logits = jax.lax.dot_general(qw, kw, (((1,), (1,)), ((), ())),
                                 preferred_element_type=jnp.float32)
    cols = jax.lax.broadcasted_iota(jnp.int32, (P2, P2), 1)
    outs = (o0, o1, o2, o3)
    for t in range(TOPK):
        m = jnp.max(logits, axis=1, keepdims=True)
        idx = jnp.min(jnp.where(logits == m, cols, P2), axis=1, keepdims=True)
        outs[t][...] = idx
        logits = jnp.where(cols == idx, -jnp.inf, logits)


def _attn_kernel(i0, i1, i2, i3, q_ref, kt0, kt1, kt2, kt3,
                 vh0, vh1, vh2, vh3, o_ref):
    kts = (kt0, kt1, kt2, kt3)
    vhs = (vh0, vh1, vh2, vh3)
    q = q_ref[0]
    outs = []
    for h in range(NUM_HEADS):
        lo, hi = h * HD, (h + 1) * HD
        qh = q[:, lo:hi]
        lgs = [
            jax.lax.dot_general(qh, kts[t][0][lo:hi, :],
                                (((1,), (0,)), ((), ())),
                                preferred_element_type=jnp.float32)
            for t in range(TOPK)]
        m = lgs[0].max(axis=1, keepdims=True)
        for t in range(1, TOPK):
            m = jnp.maximum(m, lgs[t].max(axis=1, keepdims=True))
        oh = None
        for t in range(TOPK):
            p = jnp.exp(lgs[t] - m)
            c = jax.lax.dot_general(p.astype(jnp.bfloat16),
                                    vhs[t][0, h],
                                    (((1,), (0,)), ((), ())),
                                    preferred_element_type=jnp.float32)
            oh = c if oh is None else oh + c
        outs.append(oh[:, :HD] / oh[:, HD:HD + 1])
    o_ref[...] = jnp.concatenate(outs, axis=1).reshape(WS, WS, DIM)


def _out_kernel(attn_ref, vpad_ref, lw_ref, lb_ref, wo_ref, wob_ref, o_ref):
    i = pl.program_id(0)
    acc = attn_ref[...]
    for di in range(5):
        for dj in range(5):
            w = lw_ref[di * 5 + dj:di * 5 + dj + 1, :].reshape(1, 1, DIM)
            acc = acc + vpad_ref[pl.ds(i * ROWS + di, ROWS),
                                 pl.ds(dj + 6, IMG), :] * w
    acc = acc + lb_ref[...].reshape(1, 1, DIM)
    y = jnp.dot(acc.reshape(ROWS * IMG, DIM), wo_ref[...],
                preferred_element_type=jnp.float32) + wob_ref[...]
    o_ref[...] = y.reshape(ROWS, IMG, DIM)


def kernel(x, qkv_w, qkv_b, wo_w, wo_b, lepe_w, lepe_b):
    B, H, W, C = x.shape
    f32 = jnp.float32
    q, kt, vh, vpad, qwin, kwin = pl.pallas_call(
        _qkv_kernel,
        grid=(N_WIN, N_WIN),
        in_specs=[
            pl.BlockSpec((1, WS, WS, DIM), lambda i, j: (0, i, j, 0)),
            pl.BlockSpec((DIM, 3 * DIM), lambda i, j: (0, 0)),
            pl.BlockSpec((1, 3 * DIM), lambda i, j: (0, 0)),
        ],
        out_specs=[
            pl.BlockSpec((1, W2, DIM), lambda i, j: (i * N_WIN + j, 0, 0)),
            pl.BlockSpec((1, DIM, W2), lambda i, j: (i * N_WIN + j, 0, 0)),
            pl.BlockSpec((1, NUM_HEADS, W2, HD + 1),
                         lambda i, j: (i * N_WIN + j, 0, 0, 0)),
            pl.BlockSpec((IMG + 4, 128, DIM), lambda i, j: (0, 0, 0)),
            pl.BlockSpec((1, 1, DIM), lambda i, j: (i * N_WIN + j, 0, 0)),
            pl.BlockSpec((1, 1, DIM), lambda i, j: (i * N_WIN + j, 0, 0)),
        ],
        out_shape=[
            jax.ShapeDtypeStruct((P2, W2, DIM), f32),
            jax.ShapeDtypeStruct((P2, DIM, W2), f32),
            jax.ShapeDtypeStruct((P2, NUM_HEADS, W2, HD + 1), jnp.bfloat16),
            jax.ShapeDtypeStruct((IMG + 4, 128, DIM), f32),
            jax.ShapeDtypeStruct((P2, 1, DIM), f32),
            jax.ShapeDtypeStruct((P2, 1, DIM), f32),
        ],
    )(x, qkv_w, qkv_b.reshape(1, 3 * DIM))

    o0, o1, o2, o3 = pl.pallas_call(
        _router_kernel,
        out_shape=[jax.ShapeDtypeStruct((P2, 1), jnp.int32)] * TOPK,
    )(qwin, kwin)

    def _kt_spec(t):
        return pl.BlockSpec(
            (1, DIM, W2),
            lambda p, i0, i1, i2, i3, t=t: ((i0, i1, i2, i3)[t][p, 0], 0, 0))

    def _vh_spec(t):
        return pl.BlockSpec(
            (1, NUM_HEADS, W2, HD + 1),
            lambda p, i0, i1, i2, i3, t=t: ((i0, i1, i2, i3)[t][p, 0], 0, 0, 0))

    attn_img = pl.pallas_call(
        _attn_kernel,
        grid_spec=pltpu.PrefetchScalarGridSpec(
            num_scalar_prefetch=4,
            grid=(P2,),
            in_specs=[
                pl.BlockSpec((1, W2, DIM),
                             lambda p, i0, i1, i2, i3: (p, 0, 0)),
                _kt_spec(0), _kt_spec(1), _kt_spec(2), _kt_spec(3),
                _vh_spec(0), _vh_spec(1), _vh_spec(2), _vh_spec(3),
            ],
            out_specs=pl.BlockSpec(
                (WS, WS, DIM),
                lambda p, i0, i1, i2, i3: (p // N_WIN, p % N_WIN, 0)),
        ),
        out_shape=jax.ShapeDtypeStruct((IMG, IMG, DIM), f32),
    )(o0, o1, o2, o3, q, kt, kt, kt, kt, vh, vh, vh, vh)

    out = pl.pallas_call(
        _out_kernel,
        grid=(IMG // ROWS,),
        in_specs=[
            pl.BlockSpec((ROWS, IMG, DIM), lambda i: (i, 0, 0)),
            pl.BlockSpec((IMG + 4, 128, DIM), lambda i: (0, 0, 0)),
            pl.BlockSpec((25, DIM), lambda i: (0, 0)),
            pl.BlockSpec((1, DIM), lambda i: (0, 0)),
            pl.BlockSpec((DIM, DIM), lambda i: (0, 0)),
            pl.BlockSpec((1, DIM), lambda i: (0, 0)),
        ],
        out_specs=pl.BlockSpec((ROWS, IMG, DIM), lambda i: (i, 0, 0)),
        out_shape=jax.ShapeDtypeStruct((IMG, IMG, DIM), f32),
    )(attn_img, vpad, lepe_w.reshape(25, DIM), lepe_b.reshape(1, DIM),
      wo_w, wo_b.reshape(1, DIM))

    return out[None]
